# Initial kernel scaffold; baseline (speedup 1.0000x reference)
import jax
import jax.numpy as jnp
from jax.experimental import pallas as pl


def kernel(x, edge_index, batch, Wl0, bl0, Wr0, Wl1, bl1, Wr1, Wl2, bl2, Wr2, Wl3, bl3, Wr3, Wp, bp, W1, b1, W2, b2, W3, b3):
    raise NotImplementedError("write your pallas kernel here")


# speedup vs baseline: 2.3371x; 2.3371x over previous
import functools

import jax
import jax.numpy as jnp
from jax import lax
from jax.experimental import pallas as pl
from jax.experimental.pallas import tpu as pltpu
from jax.experimental.pallas import tpu_sc as plsc

_N = 10000
_E = 320000
_D = 128
_H = 256
_G = 128
_NC = 2
_NS = 16
_NW = _NC * _NS
_CHUNK = 128
_ECH = 2560
_EPAD = _ECH * _CHUNK
_CHW = _ECH // _NW
_NPAD = 10112
_RPW = _NPAD // _NS
_BM = 400
_NBLK = _N // _BM

_f32 = jnp.float32

@functools.cache
def _sc_mesh():
  return plsc.VectorSubcoreMesh(
      core_axis_name="c", subcore_axis_name="s",
      num_cores=_NC, num_subcores=_NS)


def _agg_body(table_hbm, src_hbm, dst_hbm, zeros_hbm, out_hbm,
              src_v, dst_v, rows_v, acc_sh, sem):
  c = lax.axis_index("c")
  s = lax.axis_index("s")
  wid = s * _NC + c
  pltpu.sync_copy(zeros_hbm.at[pl.ds(s * _RPW, _RPW)],
                  acc_sh.at[pl.ds(s * _RPW, _RPW)])
  pltpu.sync_copy(src_hbm.at[pl.ds(wid * _CHW, _CHW)], src_v)
  pltpu.sync_copy(dst_hbm.at[pl.ds(wid * _CHW, _CHW)], dst_v)
  plsc.subcore_barrier()

  def body(j, carry):
    pltpu.async_copy(table_hbm.at[src_v.at[j]], rows_v, sem).wait()
    pltpu.sync_copy(rows_v, acc_sh.at[dst_v.at[j]], add=True)
    return carry

  lax.fori_loop(0, _CHW, body, 0)
  plsc.subcore_barrier()
  pltpu.sync_copy(acc_sh.at[pl.ds(s * _RPW, _RPW)],
                  out_hbm.at[c].at[pl.ds(s * _RPW, _RPW)])


@functools.cache
def _agg_call():
  return pl.kernel(
      _agg_body,
      out_type=jax.ShapeDtypeStruct((_NC, _NPAD, 128), _f32),
      mesh=_sc_mesh(),
      scratch_types=[
          pltpu.VMEM((_CHW, _CHUNK), jnp.int32),
          pltpu.VMEM((_CHW, _CHUNK), jnp.int32),
          pltpu.VMEM((_CHUNK, 128), _f32),
          pltpu.VMEM_SHARED((_NPAD, 128), _f32),
          pltpu.SemaphoreType.DMA,
      ],
  )


def _deg_body(ones_hbm, dst_hbm, zeros_hbm, out_hbm,
              dst_v, ones_v, acc_sh):
  c = lax.axis_index("c")
  s = lax.axis_index("s")
  wid = s * _NC + c
  pltpu.sync_copy(zeros_hbm.at[pl.ds(s * _RPW, _RPW)],
                  acc_sh.at[pl.ds(s * _RPW, _RPW)])
  pltpu.sync_copy(dst_hbm.at[pl.ds(wid * _CHW, _CHW)], dst_v)
  pltpu.sync_copy(ones_hbm, ones_v)
  plsc.subcore_barrier()

  def body(j, carry):
    pltpu.sync_copy(ones_v, acc_sh.at[dst_v.at[j]], add=True)
    return carry

  lax.fori_loop(0, _CHW, body, 0)
  plsc.subcore_barrier()
  pltpu.sync_copy(acc_sh.at[pl.ds(s * _RPW, _RPW)],
                  out_hbm.at[c].at[pl.ds(s * _RPW, _RPW)])


@functools.cache
def _deg_call():
  return pl.kernel(
      _deg_body,
      out_type=jax.ShapeDtypeStruct((_NC, _NPAD, 128), _f32),
      mesh=_sc_mesh(),
      scratch_types=[
          pltpu.VMEM((_CHW, _CHUNK), jnp.int32),
          pltpu.VMEM((_CHUNK, 128), _f32),
          pltpu.VMEM_SHARED((_NPAD, 128), _f32),
      ],
  )


def _inv_deg(d0_ref, d1_ref):
  deg = d0_ref[:, 0:1] + d1_ref[:, 0:1]
  return 1.0 / jnp.maximum(deg, 1.0)


def _dense0_body(x_ref, sa0_ref, sa1_ref, d0_ref, d1_ref,
                 wl_ref, wr_ref, wp_ref, bl_ref, bp_ref,
                 outa_ref, outb_ref):
  inv = _inv_deg(d0_ref, d1_ref)
  mean = (sa0_ref[...] + sa1_ref[...]) * inv
  x = x_ref[...]
  acc = jnp.dot(mean, wl_ref[...], preferred_element_type=_f32)
  acc += jnp.dot(x, wr_ref[...], preferred_element_type=_f32)
  acc += bl_ref[0:1, :]
  out = jnp.maximum(acc, 0.0)
  out += jnp.dot(x, wp_ref[...], preferred_element_type=_f32) + bp_ref[0:1, :]
  outa_ref[...] = out[:, :128]
  outb_ref[...] = out[:, 128:]


def _dense123_body(ha_ref, hb_ref, sa0_ref, sa1_ref, sb0_ref, sb1_ref,
                   d0_ref, d1_ref, wl_ref, wr_ref, bl_ref,
                   outa_ref, outb_ref, *, last):
  inv = _inv_deg(d0_ref, d1_ref)
  mean_a = (sa0_ref[...] + sa1_ref[...]) * inv
  mean_b = (sb0_ref[...] + sb1_ref[...]) * inv
  acc = jnp.dot(mean_a, wl_ref[0:128, :], preferred_element_type=_f32)
  acc += jnp.dot(mean_b, wl_ref[128:256, :], preferred_element_type=_f32)
  acc += jnp.dot(ha_ref[...], wr_ref[0:128, :], preferred_element_type=_f32)
  acc += jnp.dot(hb_ref[...], wr_ref[128:256, :], preferred_element_type=_f32)
  acc += bl_ref[0:1, :]
  out = jnp.maximum(acc, 0.0)
  if last:
    outa_ref[...] = out
  else:
    outa_ref[...] = out[:, :128]
    outb_ref[...] = out[:, 128:]


def _row_spec(w):
  return pl.BlockSpec((_BM, w), lambda m: (m, 0))


def _full_spec(r, w):
  return pl.BlockSpec((r, w), lambda m: (0, 0))


def _make_dense0():
  return pl.pallas_call(
      _dense0_body,
      grid=(_NBLK,),
      in_specs=[
          _row_spec(_D), _row_spec(128), _row_spec(128),
          _row_spec(128), _row_spec(128),
          _full_spec(_D, _H), _full_spec(_D, _H), _full_spec(_D, _H),
          _full_spec(8, _H), _full_spec(8, _H),
      ],
      out_specs=[_row_spec(128), _row_spec(128)],
      out_shape=[jax.ShapeDtypeStruct((_N, 128), _f32),
                 jax.ShapeDtypeStruct((_N, 128), _f32)],
  )


def _make_dense123(last):
  body = functools.partial(_dense123_body, last=last)
  if last:
    out_specs = [_row_spec(_H), _row_spec(128)]
    out_shape = [jax.ShapeDtypeStruct((_N, _H), _f32),
                 jax.ShapeDtypeStruct((_N, 128), _f32)]
  else:
    out_specs = [_row_spec(128), _row_spec(128)]
    out_shape = [jax.ShapeDtypeStruct((_N, 128), _f32),
                 jax.ShapeDtypeStruct((_N, 128), _f32)]
  return pl.pallas_call(
      body,
      grid=(_NBLK,),
      in_specs=[
          _row_spec(128), _row_spec(128),
          _row_spec(128), _row_spec(128), _row_spec(128), _row_spec(128),
          _row_spec(128), _row_spec(128),
          _full_spec(_H, _H), _full_spec(_H, _H), _full_spec(8, _H),
      ],
      out_specs=out_specs,
      out_shape=out_shape,
  )


def _pool_body(glo_ref, ghi_ref, onehot_ref, h_ref,
               sum_ref, cnt_ref, max_ref):
  m = pl.program_id(0)

  @pl.when(m == 0)
  def _init():
    sum_ref[...] = jnp.zeros_like(sum_ref)
    cnt_ref[...] = jnp.zeros_like(cnt_ref)
    max_ref[...] = jnp.full_like(max_ref, -3e38)

  onehot = onehot_ref[...]
  h = h_ref[...]
  dn = (((0,), (0,)), ((), ()))
  sum_ref[...] += lax.dot_general(onehot, h, dn, preferred_element_type=_f32)
  cnt_ref[...] += lax.dot_general(onehot, jnp.ones_like(h), dn,
                                  preferred_element_type=_f32)

  def gbody(g, carry):
    eg = (lax.broadcasted_iota(jnp.int32, (1, _G), 1) == g).astype(_f32)
    rmask = jnp.max(onehot * eg, axis=1, keepdims=True)
    masked = jnp.where(rmask > 0.5, h, -3e38)
    gmax = jnp.max(masked, axis=0, keepdims=True)
    cur = max_ref[pl.ds(g, 1), :]
    max_ref[pl.ds(g, 1), :] = jnp.maximum(cur, gmax)
    return carry

  lax.fori_loop(glo_ref[m], ghi_ref[m] + 1, gbody, 0)


def _make_pool():
  grid_spec = pltpu.PrefetchScalarGridSpec(
      num_scalar_prefetch=2,
      grid=(_NBLK,),
      in_specs=[
          pl.BlockSpec((_BM, _G), lambda m, *_: (m, 0)),
          pl.BlockSpec((_BM, _H), lambda m, *_: (m, 0)),
      ],
      out_specs=[
          pl.BlockSpec((_G, _H), lambda m, *_: (0, 0)),
          pl.BlockSpec((_G, _H), lambda m, *_: (0, 0)),
          pl.BlockSpec((_G, _H), lambda m, *_: (0, 0)),
      ],
  )
  return pl.pallas_call(
      _pool_body,
      grid_spec=grid_spec,
      out_shape=[jax.ShapeDtypeStruct((_G, _H), _f32)] * 3,
  )


def _mlp_body(sum_ref, cnt_ref, max_ref, w1_ref, w2_ref, w3_ref,
              b1_ref, b2_ref, b3_ref, out_ref):
  mean = sum_ref[...] / jnp.maximum(cnt_ref[...], 1.0)
  mx = max_ref[...]
  z = jnp.dot(mean, w1_ref[0:_H, :], preferred_element_type=_f32)
  z += jnp.dot(mx, w1_ref[_H:2 * _H, :], preferred_element_type=_f32)
  z = jnp.maximum(z + b1_ref[0:1, :], 0.0)
  z = jnp.dot(z, w2_ref[...], preferred_element_type=_f32) + b2_ref[0:1, :]
  z = jnp.maximum(z, 0.0)
  out_ref[...] = jnp.dot(z, w3_ref[...], preferred_element_type=_f32) \
      + b3_ref[0:1, :]


def _make_mlp():
  return pl.pallas_call(
      _mlp_body,
      out_shape=jax.ShapeDtypeStruct((_G, 128), _f32),
  )


def kernel(x, edge_index, batch, Wl0, bl0, Wr0, Wl1, bl1, Wr1, Wl2, bl2, Wr2,
           Wl3, bl3, Wr3, Wp, bp, W1, b1, W2, b2, W3, b3):
  src = edge_index[0]
  dst = edge_index[1]
  src_p = jnp.concatenate(
      [src, jnp.zeros((_EPAD - _E,), jnp.int32)]).reshape(_ECH, _CHUNK)
  dst_p = jnp.concatenate(
      [dst, jnp.full((_EPAD - _E,), _N, jnp.int32)]).reshape(_ECH, _CHUNK)
  zeros128 = jnp.zeros((_NPAD, 128), _f32)
  ones128 = jnp.ones((_CHUNK, 128), _f32)
  onehot = (batch[:, None] == jnp.arange(_G, dtype=jnp.int32)[None, :]
            ).astype(_f32)
  g_lo = batch[::_BM].astype(jnp.int32)
  g_hi = batch[_BM - 1::_BM].astype(jnp.int32)

  def pad_bias(b, w):
    return jnp.broadcast_to(b[None, :], (8, w))

  deg = _deg_call()(ones128, dst_p, zeros128)
  d0 = deg[0, :_N]
  d1 = deg[1, :_N]

  s = _agg_call()(x, src_p, dst_p, zeros128)
  ha, hb = _make_dense0()(
      x, s[0, :_N], s[1, :_N], d0, d1, Wl0, Wr0, Wp,
      pad_bias(bl0, _H), pad_bias(bp, _H))

  for i, (Wl, bl, Wr) in enumerate(
      [(Wl1, bl1, Wr1), (Wl2, bl2, Wr2), (Wl3, bl3, Wr3)]):
    last = i == 2
    sa = _agg_call()(ha, src_p, dst_p, zeros128)
    sb = _agg_call()(hb, src_p, dst_p, zeros128)
    ha, hb = _make_dense123(last)(
        ha, hb, sa[0, :_N], sa[1, :_N], sb[0, :_N], sb[1, :_N],
        d0, d1, Wl, Wr, pad_bias(bl, _H))

  h = ha
  psum, pcnt, pmax = _make_pool()(g_lo, g_hi, onehot, h)

  w3p = jnp.pad(W3, ((0, 0), (0, 128 - W3.shape[1])))
  b3p = jnp.pad(b3, (0, 128 - b3.shape[0]))
  out = _make_mlp()(psum, pcnt, pmax, W1, W2, w3p,
                    pad_bias(b1, _H), pad_bias(b2, 128), pad_bias(b3p, 128))
  return out[:, :W3.shape[1]]


# speedup vs baseline: 2.3769x; 1.0170x over previous
import functools

import jax
import jax.numpy as jnp
from jax import lax
from jax.experimental import pallas as pl
from jax.experimental.pallas import tpu as pltpu
from jax.experimental.pallas import tpu_sc as plsc

_N = 10000
_E = 320000
_D = 128
_H = 256
_G = 128
_NC = 2
_NS = 16
_NW = _NC * _NS
_CHUNK = 128
_ECH = 2560
_EPAD = _ECH * _CHUNK
_CHW = _ECH // _NW
_NPAD = 10112
_RPW = _NPAD // _NS
_BM = 400
_NBLK = _N // _BM

_f32 = jnp.float32

@functools.cache
def _sc_mesh():
  return plsc.VectorSubcoreMesh(
      core_axis_name="c", subcore_axis_name="s",
      num_cores=_NC, num_subcores=_NS)


def _agg_body(table_hbm, eidx_hbm, zeros_hbm, out_hbm,
              idx_v, rows_v, acc_sh, sem_g0, sem_g1, sem_i0, sem_i1):
  c = lax.axis_index("c")
  s = lax.axis_index("s")
  wid = s * _NC + c
  base = wid * _CHW
  pltpu.sync_copy(zeros_hbm.at[pl.ds(s * _RPW, _RPW)],
                  acc_sh.at[pl.ds(s * _RPW, _RPW)])
  plsc.subcore_barrier()

  sems_g = (sem_g0, sem_g1)
  sems_i = (sem_i0, sem_i1)
  pltpu.sync_copy(eidx_hbm.at[base], idx_v.at[0])
  pltpu.async_copy(table_hbm.at[idx_v.at[0, 0]], rows_v.at[0], sems_g[0])
  pltpu.async_copy(eidx_hbm.at[base + 1], idx_v.at[1], sems_i[1])

  def body(j2, carry):
    for b in (0, 1):
      nb = 1 - b
      j = j2 * 2 + b
      pltpu.make_async_copy(table_hbm.at[idx_v.at[b, 0]],
                            rows_v.at[b], sems_g[b]).wait()
      pltpu.make_async_copy(eidx_hbm.at[base], idx_v.at[nb],
                            sems_i[nb]).wait()
      pltpu.async_copy(table_hbm.at[idx_v.at[nb, 0]], rows_v.at[nb],
                       sems_g[nb])
      pltpu.sync_copy(rows_v.at[b], acc_sh.at[idx_v.at[b, 1]], add=True)
      pltpu.async_copy(eidx_hbm.at[base + j + 2], idx_v.at[b], sems_i[b])
    return carry

  lax.fori_loop(0, _CHW // 2, body, 0)
  pltpu.make_async_copy(table_hbm.at[idx_v.at[0, 0]], rows_v.at[0],
                        sems_g[0]).wait()
  pltpu.make_async_copy(eidx_hbm.at[base], idx_v.at[1], sems_i[1]).wait()
  plsc.subcore_barrier()
  pltpu.sync_copy(acc_sh.at[pl.ds(s * _RPW, _RPW)],
                  out_hbm.at[c].at[pl.ds(s * _RPW, _RPW)])


@functools.cache
def _agg_call():
  return pl.kernel(
      _agg_body,
      out_type=jax.ShapeDtypeStruct((_NC, _NPAD, 128), _f32),
      mesh=_sc_mesh(),
      scratch_types=[
          pltpu.VMEM((2, 2, _CHUNK), jnp.int32),
          pltpu.VMEM((2, _CHUNK, 128), _f32),
          pltpu.VMEM_SHARED((_NPAD, 128), _f32),
          pltpu.SemaphoreType.DMA,
          pltpu.SemaphoreType.DMA,
          pltpu.SemaphoreType.DMA,
          pltpu.SemaphoreType.DMA,
      ],
  )


def _deg_body(ones_hbm, dst_hbm, zeros_hbm, out_hbm,
              dst_v, ones_v, acc_sh):
  c = lax.axis_index("c")
  s = lax.axis_index("s")
  wid = s * _NC + c
  pltpu.sync_copy(zeros_hbm.at[pl.ds(s * _RPW, _RPW)],
                  acc_sh.at[pl.ds(s * _RPW, _RPW)])
  pltpu.sync_copy(dst_hbm.at[pl.ds(wid * _CHW, _CHW)], dst_v)
  pltpu.sync_copy(ones_hbm, ones_v)
  plsc.subcore_barrier()

  def body(j, carry):
    pltpu.sync_copy(ones_v, acc_sh.at[dst_v.at[j]], add=True)
    return carry

  lax.fori_loop(0, _CHW, body, 0)
  plsc.subcore_barrier()
  pltpu.sync_copy(acc_sh.at[pl.ds(s * _RPW, _RPW)],
                  out_hbm.at[c].at[pl.ds(s * _RPW, _RPW)])


@functools.cache
def _deg_call():
  return pl.kernel(
      _deg_body,
      out_type=jax.ShapeDtypeStruct((_NC, _NPAD, 128), _f32),
      mesh=_sc_mesh(),
      scratch_types=[
          pltpu.VMEM((_CHW, _CHUNK), jnp.int32),
          pltpu.VMEM((_CHUNK, 128), _f32),
          pltpu.VMEM_SHARED((_NPAD, 128), _f32),
      ],
  )


def _inv_deg(d0_ref, d1_ref):
  deg = d0_ref[:, 0:1] + d1_ref[:, 0:1]
  return 1.0 / jnp.maximum(deg, 1.0)


def _dense0_body(x_ref, sa0_ref, sa1_ref, d0_ref, d1_ref,
                 wl_ref, wr_ref, wp_ref, bl_ref, bp_ref,
                 outa_ref, outb_ref):
  inv = _inv_deg(d0_ref, d1_ref)
  mean = (sa0_ref[...] + sa1_ref[...]) * inv
  x = x_ref[...]
  acc = jnp.dot(mean, wl_ref[...], preferred_element_type=_f32)
  acc += jnp.dot(x, wr_ref[...], preferred_element_type=_f32)
  acc += bl_ref[0:1, :]
  out = jnp.maximum(acc, 0.0)
  out += jnp.dot(x, wp_ref[...], preferred_element_type=_f32) + bp_ref[0:1, :]
  outa_ref[...] = out[:, :128]
  outb_ref[...] = out[:, 128:]


def _dense123_body(ha_ref, hb_ref, sa0_ref, sa1_ref, sb0_ref, sb1_ref,
                   d0_ref, d1_ref, wl_ref, wr_ref, bl_ref,
                   outa_ref, outb_ref, *, last):
  inv = _inv_deg(d0_ref, d1_ref)
  mean_a = (sa0_ref[...] + sa1_ref[...]) * inv
  mean_b = (sb0_ref[...] + sb1_ref[...]) * inv
  acc = jnp.dot(mean_a, wl_ref[0:128, :], preferred_element_type=_f32)
  acc += jnp.dot(mean_b, wl_ref[128:256, :], preferred_element_type=_f32)
  acc += jnp.dot(ha_ref[...], wr_ref[0:128, :], preferred_element_type=_f32)
  acc += jnp.dot(hb_ref[...], wr_ref[128:256, :], preferred_element_type=_f32)
  acc += bl_ref[0:1, :]
  out = jnp.maximum(acc, 0.0)
  if last:
    outa_ref[...] = out
  else:
    outa_ref[...] = out[:, :128]
    outb_ref[...] = out[:, 128:]


def _row_spec(w):
  return pl.BlockSpec((_BM, w), lambda m: (m, 0))


def _full_spec(r, w):
  return pl.BlockSpec((r, w), lambda m: (0, 0))


def _make_dense0():
  return pl.pallas_call(
      _dense0_body,
      grid=(_NBLK,),
      in_specs=[
          _row_spec(_D), _row_spec(128), _row_spec(128),
          _row_spec(128), _row_spec(128),
          _full_spec(_D, _H), _full_spec(_D, _H), _full_spec(_D, _H),
          _full_spec(8, _H), _full_spec(8, _H),
      ],
      out_specs=[_row_spec(128), _row_spec(128)],
      out_shape=[jax.ShapeDtypeStruct((_N, 128), _f32),
                 jax.ShapeDtypeStruct((_N, 128), _f32)],
  )


def _make_dense123(last):
  body = functools.partial(_dense123_body, last=last)
  if last:
    out_specs = [_row_spec(_H), _row_spec(128)]
    out_shape = [jax.ShapeDtypeStruct((_N, _H), _f32),
                 jax.ShapeDtypeStruct((_N, 128), _f32)]
  else:
    out_specs = [_row_spec(128), _row_spec(128)]
    out_shape = [jax.ShapeDtypeStruct((_N, 128), _f32),
                 jax.ShapeDtypeStruct((_N, 128), _f32)]
  return pl.pallas_call(
      body,
      grid=(_NBLK,),
      in_specs=[
          _row_spec(128), _row_spec(128),
          _row_spec(128), _row_spec(128), _row_spec(128), _row_spec(128),
          _row_spec(128), _row_spec(128),
          _full_spec(_H, _H), _full_spec(_H, _H), _full_spec(8, _H),
      ],
      out_specs=out_specs,
      out_shape=out_shape,
  )


def _pool_body(glo_ref, ghi_ref, onehot_ref, h_ref,
               sum_ref, cnt_ref, max_ref):
  m = pl.program_id(0)

  @pl.when(m == 0)
  def _init():
    sum_ref[...] = jnp.zeros_like(sum_ref)
    cnt_ref[...] = jnp.zeros_like(cnt_ref)
    max_ref[...] = jnp.full_like(max_ref, -3e38)

  onehot = onehot_ref[...]
  h = h_ref[...]
  dn = (((0,), (0,)), ((), ()))
  sum_ref[...] += lax.dot_general(onehot, h, dn, preferred_element_type=_f32)
  cnt_ref[...] += lax.dot_general(onehot, jnp.ones_like(h), dn,
                                  preferred_element_type=_f32)

  def gbody(g, carry):
    eg = (lax.broadcasted_iota(jnp.int32, (1, _G), 1) == g).astype(_f32)
    rmask = jnp.max(onehot * eg, axis=1, keepdims=True)
    masked = jnp.where(rmask > 0.5, h, -3e38)
    gmax = jnp.max(masked, axis=0, keepdims=True)
    cur = max_ref[pl.ds(g, 1), :]
    max_ref[pl.ds(g, 1), :] = jnp.maximum(cur, gmax)
    return carry

  lax.fori_loop(glo_ref[m], ghi_ref[m] + 1, gbody, 0)


def _make_pool():
  grid_spec = pltpu.PrefetchScalarGridSpec(
      num_scalar_prefetch=2,
      grid=(_NBLK,),
      in_specs=[
          pl.BlockSpec((_BM, _G), lambda m, *_: (m, 0)),
          pl.BlockSpec((_BM, _H), lambda m, *_: (m, 0)),
      ],
      out_specs=[
          pl.BlockSpec((_G, _H), lambda m, *_: (0, 0)),
          pl.BlockSpec((_G, _H), lambda m, *_: (0, 0)),
          pl.BlockSpec((_G, _H), lambda m, *_: (0, 0)),
      ],
  )
  return pl.pallas_call(
      _pool_body,
      grid_spec=grid_spec,
      out_shape=[jax.ShapeDtypeStruct((_G, _H), _f32)] * 3,
  )


def _mlp_body(sum_ref, cnt_ref, max_ref, w1_ref, w2_ref, w3_ref,
              b1_ref, b2_ref, b3_ref, out_ref):
  mean = sum_ref[...] / jnp.maximum(cnt_ref[...], 1.0)
  mx = max_ref[...]
  z = jnp.dot(mean, w1_ref[0:_H, :], preferred_element_type=_f32)
  z += jnp.dot(mx, w1_ref[_H:2 * _H, :], preferred_element_type=_f32)
  z = jnp.maximum(z + b1_ref[0:1, :], 0.0)
  z = jnp.dot(z, w2_ref[...], preferred_element_type=_f32) + b2_ref[0:1, :]
  z = jnp.maximum(z, 0.0)
  out_ref[...] = jnp.dot(z, w3_ref[...], preferred_element_type=_f32) \
      + b3_ref[0:1, :]


def _make_mlp():
  return pl.pallas_call(
      _mlp_body,
      out_shape=jax.ShapeDtypeStruct((_G, 128), _f32),
  )


def kernel(x, edge_index, batch, Wl0, bl0, Wr0, Wl1, bl1, Wr1, Wl2, bl2, Wr2,
           Wl3, bl3, Wr3, Wp, bp, W1, b1, W2, b2, W3, b3):
  src = edge_index[0]
  dst = edge_index[1]
  src_p = jnp.concatenate(
      [src, jnp.zeros((_EPAD - _E,), jnp.int32)]).reshape(_ECH, _CHUNK)
  dst_p = jnp.concatenate(
      [dst, jnp.full((_EPAD - _E,), _N, jnp.int32)]).reshape(_ECH, _CHUNK)
  eidx = jnp.concatenate(
      [jnp.stack([src_p, dst_p], axis=1),
       jnp.zeros((2, 2, _CHUNK), jnp.int32)], axis=0)
  zeros128 = jnp.zeros((_NPAD, 128), _f32)
  ones128 = jnp.ones((_CHUNK, 128), _f32)
  onehot = (batch[:, None] == jnp.arange(_G, dtype=jnp.int32)[None, :]
            ).astype(_f32)
  g_lo = batch[::_BM].astype(jnp.int32)
  g_hi = batch[_BM - 1::_BM].astype(jnp.int32)

  def pad_bias(b, w):
    return jnp.broadcast_to(b[None, :], (8, w))

  deg = _deg_call()(ones128, dst_p, zeros128)
  d0 = deg[0, :_N]
  d1 = deg[1, :_N]

  s = _agg_call()(x, eidx, zeros128)
  ha, hb = _make_dense0()(
      x, s[0, :_N], s[1, :_N], d0, d1, Wl0, Wr0, Wp,
      pad_bias(bl0, _H), pad_bias(bp, _H))

  for i, (Wl, bl, Wr) in enumerate(
      [(Wl1, bl1, Wr1), (Wl2, bl2, Wr2), (Wl3, bl3, Wr3)]):
    last = i == 2
    sa = _agg_call()(ha, eidx, zeros128)
    sb = _agg_call()(hb, eidx, zeros128)
    ha, hb = _make_dense123(last)(
        ha, hb, sa[0, :_N], sa[1, :_N], sb[0, :_N], sb[1, :_N],
        d0, d1, Wl, Wr, pad_bias(bl, _H))

  h = ha
  psum, pcnt, pmax = _make_pool()(g_lo, g_hi, onehot, h)

  w3p = jnp.pad(W3, ((0, 0), (0, 128 - W3.shape[1])))
  b3p = jnp.pad(b3, (0, 128 - b3.shape[0]))
  out = _make_mlp()(psum, pcnt, pmax, W1, W2, w3p,
                    pad_bias(b1, _H), pad_bias(b2, 128), pad_bias(b3p, 128))
  return out[:, :W3.shape[1]]
